# baseline (speedup 1.0000x reference)
import functools

import jax
import jax.numpy as jnp
from jax import lax
from jax.experimental import pallas as pl
from jax.experimental.pallas import tpu as pltpu
from jax.experimental.pallas import tpu_sc as plsc

_NUM_CORES = 2
_NUM_SUBCORES = 16
_NW = _NUM_CORES * _NUM_SUBCORES
_LANES = 16


def _make_sc_bag_sum(B, Lb, D, table_rows, va, *, cb=2, nbuf=4,
                     interpret=False):
  assert B % _NW == 0
  bpw = B // _NW
  assert bpw % cb == 0
  nchunk = bpw // cb
  assert nchunk % nbuf == 0
  assert D % _LANES == 0
  nq = D // _LANES
  ngrp = (Lb + _LANES - 1) // _LANES
  nnz_w = bpw * Lb
  offs = []
  o = 0
  while o + _LANES < Lb:
    offs.append(o)
    o += _LANES
  offs.append(Lb - _LANES)

  mesh = plsc.VectorSubcoreMesh(
      core_axis_name="c", subcore_axis_name="s",
      num_cores=_NUM_CORES, num_subcores=_NUM_SUBCORES)

  def body(idx_hbm, w_hbm, table_hbm, out_hbm,
           idx_v, w_v, gidx_v, rows_v, out_v, *sems):
    cid = lax.axis_index("c")
    sid = lax.axis_index("s")
    wid = sid * _NUM_CORES + cid
    gbase = wid * nnz_w
    bag0 = wid * bpw

    pltpu.sync_copy(idx_hbm.at[pl.ds(gbase, nnz_w)],
                    idx_v.at[pl.ds(0, nnz_w)])
    pltpu.sync_copy(w_hbm.at[pl.ds(gbase, nnz_w)],
                    w_v.at[pl.ds(0, nnz_w)])

    def prep(chunk, b):
      for k in range(cb):
        woff = (chunk * cb + k) * Lb
        for o in offs:
          iv = idx_v[pl.ds(woff + o, _LANES)]
          gidx_v[b, k, pl.ds(o, _LANES)] = jnp.where(
              iv < va, iv * 2, (iv - va) * 2 + 1)

    def gather_copies(b):
      return [
          pltpu.make_async_copy(
              table_hbm.at[gidx_v.at[b, k]],
              rows_v.at[b, pl.ds(k * Lb, Lb)],
              sems[b])
          for k in range(cb)
      ]

    def start(chunk, b):
      prep(chunk, b)
      for d in gather_copies(b):
        d.start()

    def drain(b):
      for d in gather_copies(b):
        d.wait()

    for b in range(nbuf):
      start(b, b)

    def outer(i, carry):
      for b in range(nbuf):
        chunk = i * nbuf + b
        drain(b)
        for k in range(cb):
          woff = (chunk * cb + k) * Lb
          acc = [jnp.zeros((_LANES,), jnp.float32) for _ in range(nq)]
          for g in range(ngrp):
            nrows = min(_LANES, Lb - g * _LANES)
            wvec = w_v[pl.ds(woff + g * _LANES, _LANES)]
            for j2 in range(nrows):
              wv = jnp.full((_LANES,), wvec[j2])
              r = k * Lb + g * _LANES + j2
              for q in range(nq):
                acc[q] = acc[q] + wv * rows_v[b, r, pl.ds(q * _LANES, _LANES)]
          for q in range(nq):
            out_v[k, pl.ds(q * _LANES, _LANES)] = acc[q]
        pltpu.sync_copy(out_v, out_hbm.at[pl.ds(bag0 + chunk * cb, cb)])
        nxt = chunk + nbuf

        @pl.when(nxt < nchunk)
        def _():
          start(nxt, b)
      return carry

    lax.fori_loop(0, nchunk // nbuf, outer, 0)

  fn = pl.kernel(
      body,
      out_type=jax.ShapeDtypeStruct((B, D), jnp.float32),
      mesh=mesh,
      scratch_types=[
          pltpu.VMEM((nnz_w + _LANES,), jnp.int32),
          pltpu.VMEM((nnz_w + _LANES,), jnp.float32),
          pltpu.VMEM((nbuf, cb, Lb), jnp.int32),
          pltpu.VMEM((nbuf, cb * Lb, D), jnp.float32),
          pltpu.VMEM((cb, D), jnp.float32),
      ] + [pltpu.SemaphoreType.DMA] * nbuf,
      compiler_params=pltpu.CompilerParams(use_tc_tiling_on_sc=False),
      interpret=interpret,
  )
  return fn


def _make_tc_format(V, E, VA, *, blkc=6400, interpret=False):
  assert VA % blkc == 0 and blkc % 128 == 0
  nblk = VA // blkc
  last_blk = (V - 1) // blkc

  def body(a_ref, b_ref, o_ref):
    eye = jnp.eye(E, dtype=jnp.float32)
    o_ref[:, pl.ds(0, E)] = jnp.transpose(a_ref[...])
    o_ref[:, pl.ds(E, E)] = lax.dot_general(
        b_ref[...], eye, (((0,), (0,)), ((), ())),
        precision=lax.Precision.HIGHEST,
        preferred_element_type=jnp.float32)

  return pl.pallas_call(
      body,
      grid=(nblk,),
      in_specs=[
          pl.BlockSpec((E, blkc), lambda j: (0, j)),
          pl.BlockSpec((E, blkc),
                       lambda j: (0, jnp.minimum(j + nblk, last_blk))),
      ],
      out_specs=pl.BlockSpec((blkc, 2 * E), lambda j: (j, 0)),
      out_shape=jax.ShapeDtypeStruct((VA, 2 * E), jnp.float32),
      interpret=interpret,
  )


def _make_tc_mlp(B, D, E, *, blk=1024, interpret=False):
  assert B % blk == 0

  def body(x_ref, w1_ref, b1_ref, w2_ref, b2_ref, o_ref):
    x = x_ref[...]
    s = jnp.sum(x * x, axis=1, keepdims=True)
    x = x / jnp.maximum(jnp.sqrt(s), 1e-12)
    ht = lax.dot_general(w1_ref[...], x, (((1,), (1,)), ((), ())),
                         preferred_element_type=jnp.float32) + b1_ref[...]
    ht = jnp.where(ht >= 0, ht, 0.01 * ht)
    o_ref[...] = lax.dot_general(w2_ref[...], ht, (((1,), (0,)), ((), ())),
                                 preferred_element_type=jnp.float32) + b2_ref[...]

  grid = (B // blk,)
  return pl.pallas_call(
      body,
      grid=grid,
      in_specs=[
          pl.BlockSpec((blk, E), lambda i: (i, 0)),
          pl.BlockSpec((D, E), lambda i: (0, 0)),
          pl.BlockSpec((D, 1), lambda i: (0, 0)),
          pl.BlockSpec((D, D), lambda i: (0, 0)),
          pl.BlockSpec((D, 1), lambda i: (0, 0)),
      ],
      out_specs=pl.BlockSpec((D, blk), lambda i: (0, i)),
      out_shape=jax.ShapeDtypeStruct((D, B), jnp.float32),
      interpret=interpret,
  )


@jax.jit
def kernel(indices, offsets, weights, base_emb, W1, b1, W2, b2):
  del offsets
  B = 16384
  Lb = 50
  V, E = base_emb.shape
  D = W1.shape[0]
  VA = 512000
  fmt = _make_tc_format(V, E, VA)
  table_lin = fmt(base_emb.T, base_emb.T).reshape(2 * VA, E)
  sc = _make_sc_bag_sum(B, Lb, E, 2 * VA, VA)
  bag_sums = sc(indices, weights, table_lin)
  mlp = _make_tc_mlp(B, D, E)
  yt = mlp(bag_sums, W1, b1.reshape(D, 1), W2, b2.reshape(D, 1))
  return yt.T


# speedup vs baseline: 1.0278x; 1.0278x over previous
import functools

import jax
import jax.numpy as jnp
from jax import lax
from jax.experimental import pallas as pl
from jax.experimental.pallas import tpu as pltpu
from jax.experimental.pallas import tpu_sc as plsc

_NUM_CORES = 2
_NUM_SUBCORES = 16
_NW = _NUM_CORES * _NUM_SUBCORES
_LANES = 16


def _make_sc_bag_sum(B, Lb, D, table_rows, va, *, cb=2, nbuf=2,
                     interpret=False):
  assert B % _NW == 0
  bpw = B // _NW
  assert bpw % cb == 0
  nchunk = bpw // cb
  assert nchunk % nbuf == 0
  assert D % _LANES == 0
  nq = D // _LANES
  ngrp = (Lb + _LANES - 1) // _LANES
  nnz_w = bpw * Lb
  offs = []
  o = 0
  while o + _LANES < Lb:
    offs.append(o)
    o += _LANES
  offs.append(Lb - _LANES)

  mesh = plsc.VectorSubcoreMesh(
      core_axis_name="c", subcore_axis_name="s",
      num_cores=_NUM_CORES, num_subcores=_NUM_SUBCORES)

  def body(idx_hbm, w_hbm, table_hbm, out_hbm,
           idx_v, w_v, gidx_v, rows_v, out_v, *sems):
    osems = sems[nbuf:]
    sems = sems[:nbuf]
    cid = lax.axis_index("c")
    sid = lax.axis_index("s")
    wid = sid * _NUM_CORES + cid
    gbase = wid * nnz_w
    bag0 = wid * bpw

    pltpu.sync_copy(idx_hbm.at[pl.ds(gbase, nnz_w)],
                    idx_v.at[pl.ds(0, nnz_w)])
    pltpu.sync_copy(w_hbm.at[pl.ds(gbase, nnz_w)],
                    w_v.at[pl.ds(0, nnz_w)])

    def prep(chunk, b):
      for k in range(cb):
        woff = (chunk * cb + k) * Lb
        for o in offs:
          iv = idx_v[pl.ds(woff + o, _LANES)]
          gidx_v[b, k, pl.ds(o, _LANES)] = jnp.where(
              iv < va, iv * 2, (iv - va) * 2 + 1)

    def gather_copies(b):
      return [
          pltpu.make_async_copy(
              table_hbm.at[gidx_v.at[b, k]],
              rows_v.at[b, pl.ds(k * Lb, Lb)],
              sems[b])
          for k in range(cb)
      ]

    def start(chunk, b):
      prep(chunk, b)
      for d in gather_copies(b):
        d.start()

    def drain(b):
      for d in gather_copies(b):
        d.wait()

    for b in range(nbuf):
      start(b, b)

    def outer(i, carry):
      for b in range(nbuf):
        chunk = i * nbuf + b
        drain(b)

        @pl.when(chunk >= nbuf)
        def _():
          pltpu.make_async_copy(
              out_v.at[b],
              out_hbm.at[pl.ds(bag0 + (chunk - nbuf) * cb, cb)],
              osems[b]).wait()

        for k in range(cb):
          woff = (chunk * cb + k) * Lb
          acc = [jnp.zeros((_LANES,), jnp.float32) for _ in range(nq)]
          for g in range(ngrp):
            nrows = min(_LANES, Lb - g * _LANES)
            wvec = w_v[pl.ds(woff + g * _LANES, _LANES)]
            for j2 in range(nrows):
              wv = jnp.full((_LANES,), wvec[j2])
              r = k * Lb + g * _LANES + j2
              for q in range(nq):
                acc[q] = acc[q] + wv * rows_v[b, r, pl.ds(q * _LANES, _LANES)]
          for q in range(nq):
            out_v[b, k, pl.ds(q * _LANES, _LANES)] = acc[q]
        pltpu.make_async_copy(
            out_v.at[b], out_hbm.at[pl.ds(bag0 + chunk * cb, cb)],
            osems[b]).start()
        nxt = chunk + nbuf

        @pl.when(nxt < nchunk)
        def _():
          start(nxt, b)
      return carry

    lax.fori_loop(0, nchunk // nbuf, outer, 0)

    for b in range(nbuf):
      pltpu.make_async_copy(
          out_v.at[b],
          out_hbm.at[pl.ds(bag0 + (nchunk - nbuf + b) * cb, cb)],
          osems[b]).wait()

  fn = pl.kernel(
      body,
      out_type=jax.ShapeDtypeStruct((B, D), jnp.float32),
      mesh=mesh,
      scratch_types=[
          pltpu.VMEM((nnz_w + _LANES,), jnp.int32),
          pltpu.VMEM((nnz_w + _LANES,), jnp.float32),
          pltpu.VMEM((nbuf, cb, Lb), jnp.int32),
          pltpu.VMEM((nbuf, cb * Lb, D), jnp.float32),
          pltpu.VMEM((nbuf, cb, D), jnp.float32),
      ] + [pltpu.SemaphoreType.DMA] * (2 * nbuf),
      compiler_params=pltpu.CompilerParams(use_tc_tiling_on_sc=False),
      interpret=interpret,
  )
  return fn


def _make_tc_format(V, E, VA, *, blkc=6400, interpret=False):
  assert VA % blkc == 0 and blkc % 128 == 0
  nblk = VA // blkc
  last_blk = (V - 1) // blkc

  def body(a_ref, b_ref, o_ref):
    eye = jnp.eye(E, dtype=jnp.float32)
    o_ref[:, pl.ds(0, E)] = jnp.transpose(a_ref[...])
    o_ref[:, pl.ds(E, E)] = lax.dot_general(
        b_ref[...], eye, (((0,), (0,)), ((), ())),
        precision=lax.Precision.HIGHEST,
        preferred_element_type=jnp.float32)

  return pl.pallas_call(
      body,
      grid=(nblk,),
      in_specs=[
          pl.BlockSpec((E, blkc), lambda j: (0, j)),
          pl.BlockSpec((E, blkc),
                       lambda j: (0, jnp.minimum(j + nblk, last_blk))),
      ],
      out_specs=pl.BlockSpec((blkc, 2 * E), lambda j: (j, 0)),
      out_shape=jax.ShapeDtypeStruct((VA, 2 * E), jnp.float32),
      interpret=interpret,
  )


def _make_tc_mlp(B, D, E, *, blk=1024, interpret=False):
  assert B % blk == 0

  def body(x_ref, w1_ref, b1_ref, w2_ref, b2_ref, o_ref):
    x = x_ref[...]
    s = jnp.sum(x * x, axis=1, keepdims=True)
    x = x / jnp.maximum(jnp.sqrt(s), 1e-12)
    ht = lax.dot_general(w1_ref[...], x, (((1,), (1,)), ((), ())),
                         preferred_element_type=jnp.float32) + b1_ref[...]
    ht = jnp.where(ht >= 0, ht, 0.01 * ht)
    o_ref[...] = lax.dot_general(w2_ref[...], ht, (((1,), (0,)), ((), ())),
                                 preferred_element_type=jnp.float32) + b2_ref[...]

  grid = (B // blk,)
  return pl.pallas_call(
      body,
      grid=grid,
      in_specs=[
          pl.BlockSpec((blk, E), lambda i: (i, 0)),
          pl.BlockSpec((D, E), lambda i: (0, 0)),
          pl.BlockSpec((D, 1), lambda i: (0, 0)),
          pl.BlockSpec((D, D), lambda i: (0, 0)),
          pl.BlockSpec((D, 1), lambda i: (0, 0)),
      ],
      out_specs=pl.BlockSpec((D, blk), lambda i: (0, i)),
      out_shape=jax.ShapeDtypeStruct((D, B), jnp.float32),
      interpret=interpret,
  )


@jax.jit
def kernel(indices, offsets, weights, base_emb, W1, b1, W2, b2):
  del offsets
  B = 16384
  Lb = 50
  V, E = base_emb.shape
  D = W1.shape[0]
  VA = 512000
  fmt = _make_tc_format(V, E, VA)
  table_lin = fmt(base_emb.T, base_emb.T).reshape(2 * VA, E)
  sc = _make_sc_bag_sum(B, Lb, E, 2 * VA, VA)
  bag_sums = sc(indices, weights, table_lin)
  mlp = _make_tc_mlp(B, D, E)
  yt = mlp(bag_sums, W1, b1.reshape(D, 1), W2, b2.reshape(D, 1))
  return yt.T
